# Initial kernel scaffold; baseline (speedup 1.0000x reference)
import jax
import jax.numpy as jnp
from jax.experimental import pallas as pl


def kernel(pos, edge_index, batch, W1a, b1a, W1b, b1b, W2a, b2a, W2b, b2b):
    raise NotImplementedError("write your pallas kernel here")


# speedup vs baseline: 1.6671x; 1.6671x over previous
import jax
import jax.numpy as jnp
from jax import lax
from jax.experimental import pallas as pl
from jax.experimental.pallas import tpu as pltpu
from jax.experimental.pallas import tpu_sc as plsc

N = 10000
E = 160000
NC = 2
NS = 16
NW = NC * NS
L = 16
ER = E // L

EPT_G = 4992
ROWS_G = EPT_G // L
TAIL_BASE = NW * EPT_G

EPT = E // NW
GB = 64

RPT = 320
NPAD = NW * RPT
CE = 4096
NCHUNK = E // CE
TE = E - NCHUNK * CE
FG = 128
MCAP = CE + 2 * FG
CAPL = E + FG
WB = 40
WCAP = WB * GB
SCAP = 4352
HCAP = 384

_mesh = plsc.VectorSubcoreMesh(core_axis_name="c", subcore_axis_name="s")
_sc_params = pltpu.CompilerParams(needs_layout_passes=False)


def _wid():
    return lax.axis_index("s") * NC + lax.axis_index("c")


def _iota():
    return lax.iota(jnp.int32, L)


def _geom_range(src, dst, posx, posy, posz, ef_out,
                sidx, didx, sxb, dxb, efbuf, sem, base, cnt):
    pltpu.sync_copy(src.at[pl.ds(base, cnt)], sidx.at[pl.ds(0, cnt)])
    pltpu.sync_copy(dst.at[pl.ds(base, cnt)], didx.at[pl.ds(0, cnt)])
    for c, table in enumerate((posx, posy, posz)):
        pltpu.async_copy(table.at[sidx.at[pl.ds(0, cnt)]],
                         sxb.at[pl.ds(0, cnt)], sem).wait()
        pltpu.async_copy(table.at[didx.at[pl.ds(0, cnt)]],
                         dxb.at[pl.ds(0, cnt)], sem).wait()

        def body(v, _):
            sv = sxb[pl.ds(v * L, L)]
            dv = dxb[pl.ds(v * L, L)]
            efbuf[c, pl.ds(v * L, L)] = sv
            efbuf[3 + c, pl.ds(v * L, L)] = sv - dv
            return 0

        lax.fori_loop(0, cnt // L, body, 0)
    pltpu.sync_copy(efbuf.at[:, pl.ds(0, cnt)],
                    ef_out.at[:, pl.ds(base, cnt)])


def _geom_body(posx, posy, posz, src, dst, ef_out,
               sidx, didx, sxb, dxb, efbuf, sem):
    w = _wid()
    _geom_range(src, dst, posx, posy, posz, ef_out,
                sidx, didx, sxb, dxb, efbuf, sem, w * EPT_G, EPT_G)

    @pl.when(w < 2)
    def _():
        _geom_range(src, dst, posx, posy, posz, ef_out,
                    sidx, didx, sxb, dxb, efbuf, sem,
                    TAIL_BASE + w * 128, 128)


def _sc_geom(posx, posy, posz, src, dst):
    f = pl.kernel(
        _geom_body,
        out_type=jax.ShapeDtypeStruct((6, E), jnp.float32),
        mesh=_mesh,
        compiler_params=_sc_params,
        scratch_types=[
            pltpu.VMEM((EPT_G,), jnp.int32),
            pltpu.VMEM((EPT_G,), jnp.int32),
            pltpu.VMEM((EPT_G,), jnp.float32),
            pltpu.VMEM((EPT_G,), jnp.float32),
            pltpu.VMEM((6, EPT_G), jnp.float32),
            pltpu.SemaphoreType.DMA,
        ],
    )
    return f(posx, posy, posz, src, dst)


def _gather_body(table, src, out, idxbuf, rb0, rb1, gs0, gs1, os0, os1):
    w = _wid()
    base = w * EPT
    nfull = EPT // GB
    tail = EPT - nfull * GB
    pltpu.sync_copy(src.at[pl.ds(base, EPT + GB)], idxbuf)
    rbufs = (rb0, rb1)
    gsems = (gs0, gs1)
    osems = (os0, os1)

    def issue(g, slot):
        pltpu.async_copy(table.at[idxbuf.at[pl.ds(g * GB, GB)]],
                         rbufs[slot], gsems[slot])

    def wait_gather(slot):
        pltpu.make_async_copy(table.at[idxbuf.at[pl.ds(0, GB)]],
                              rbufs[slot], gsems[slot]).wait()

    def copy_out(g, slot):
        pltpu.async_copy(rbufs[slot], out.at[pl.ds(base + g * GB, GB)],
                         osems[slot])

    def wait_out(slot):
        pltpu.make_async_copy(rbufs[slot], out.at[pl.ds(base, GB)],
                              osems[slot]).wait()

    issue(0, 0)

    def body(g, _):
        even = lax.rem(g, 2) == 0

        @pl.when((g + 1 < nfull) & even)
        def _():
            @pl.when(g + 1 >= 2)
            def _():
                wait_out(1)
            issue(g + 1, 1)

        @pl.when((g + 1 < nfull) & jnp.logical_not(even))
        def _():
            @pl.when(g + 1 >= 2)
            def _():
                wait_out(0)
            issue(g + 1, 0)

        @pl.when(even)
        def _():
            wait_gather(0)
            copy_out(g, 0)

        @pl.when(jnp.logical_not(even))
        def _():
            wait_gather(1)
            copy_out(g, 1)

        return 0

    lax.fori_loop(0, nfull, body, 0)
    wait_out((nfull - 2) % 2)
    wait_out((nfull - 1) % 2)
    pltpu.async_copy(table.at[idxbuf.at[pl.ds(nfull * GB, GB)]], rb0,
                     gs0).wait()
    pltpu.sync_copy(rb0.at[pl.ds(0, tail)],
                    out.at[pl.ds(base + nfull * GB, tail)])


def _sc_gather(table, src):
    f = pl.kernel(
        _gather_body,
        out_type=jax.ShapeDtypeStruct((E, 256), jnp.float32),
        mesh=_mesh,
        compiler_params=_sc_params,
        scratch_types=[
            pltpu.VMEM((EPT + GB,), jnp.int32),
            pltpu.VMEM((GB, 256), jnp.float32),
            pltpu.VMEM((GB, 256), jnp.float32),
            pltpu.SemaphoreType.DMA,
            pltpu.SemaphoreType.DMA,
            pltpu.SemaphoreType.DMA,
            pltpu.SemaphoreType.DMA,
        ],
    )
    return f(table, src)


def _partition_body(dst, lists, counts, dstbuf, pkbuf, cntbuf, sbuf, hist, sem):
    w = _wid()
    lo = w * RPT
    lbase = w * CAPL
    junk = jnp.full((L,), RPT << 18, jnp.int32)

    def do_chunk(ebase, ne, state):
        gp, carry = state
        pltpu.sync_copy(dst.at[pl.ds(ebase, ne)], dstbuf.at[pl.ds(0, ne)])
        base_eid = jnp.full((L,), ebase, jnp.int32) + _iota()

        def scan_v(v, off):
            d = dstbuf[pl.ds(v * L, L)]
            m = (d >= lo) & (d < lo + RPT)
            pk = ((d - lo) << 18) | (base_eid + v * L)
            plsc.store_compressed(pkbuf.at[pl.ds(off, L)], pk, mask=m)
            cnt = plsc.all_reduce_population_count(m)
            return off + cnt[0]

        total = lax.fori_loop(0, ne // L, scan_v, carry)
        nfull = total // FG

        def flush(b, _):
            o = pl.multiple_of(lbase + gp + b * FG, FG)
            pltpu.sync_copy(pkbuf.at[pl.ds(b * FG, FG)],
                            lists.at[pl.ds(o, FG)])
            return 0

        lax.fori_loop(0, nfull, flush, 0)
        tsrc = nfull * FG
        for t in range(FG // L):
            v = pkbuf[pl.ds(tsrc + t * L, L)]
            pkbuf[pl.ds(t * L, L)] = v
        return gp + nfull * FG, total - nfull * FG

    def chunk_body(c, state):
        return do_chunk(c * CE, CE, state)

    state = lax.fori_loop(0, NCHUNK, chunk_body, (0, 0))
    gp, carry = do_chunk(NCHUNK * CE, TE, state)

    for t in range(FG // L):
        pkbuf[pl.ds(carry + t * L, L)] = junk
    pltpu.sync_copy(pkbuf.at[pl.ds(0, FG)],
                    lists.at[pl.ds(pl.multiple_of(lbase + gp, FG), FG)])
    count = gp + carry
    cvec = jnp.full((L,), 0, jnp.int32) + count
    for t in range(FG // L):
        cntbuf[pl.ds(t * L, L)] = cvec
    pltpu.sync_copy(cntbuf, counts.at[pl.ds(FG * w, FG)])

    onehot0 = jnp.where(_iota() == 0, 1, 0).astype(jnp.int32)
    lane0 = _iota() == 0
    pl_len = ((count + FG - 1) // FG) * FG
    nchb = (pl_len + SCAP - 1) // SCAP

    def sort_chunk(c, _):
        cbase = c * SCAP
        o = pl.multiple_of(lbase + cbase, FG)
        pltpu.sync_copy(lists.at[pl.ds(o, SCAP)], pkbuf.at[pl.ds(0, SCAP)])
        chunk_n = jnp.clip(count - cbase, 0, SCAP)

        def prefill(v, _):
            sbuf[pl.ds(v * L, L)] = junk
            return 0

        lax.fori_loop(0, SCAP // L, prefill, 0)

        def hzero(v, _):
            hist[pl.ds(v * L, L)] = jnp.zeros((L,), jnp.int32)
            return 0

        lax.fori_loop(0, HCAP // L, hzero, 0)

        def hcount(i, _):
            pk = pkbuf[pl.ds(i, L)][0]
            loc = pk >> 18
            hist[pl.ds(loc, L)] = hist[pl.ds(loc, L)] + onehot0
            return 0

        lax.fori_loop(0, chunk_n, hcount, 0)

        def prefix(b, run):
            cnt = hist[pl.ds(b, L)][0]
            plsc.store_compressed(hist.at[pl.ds(b, L)],
                                  jnp.full((L,), 0, jnp.int32) + run,
                                  mask=lane0)
            return run + cnt

        lax.fori_loop(0, RPT, prefix, 0)

        def place(i, _):
            pk = pkbuf[pl.ds(i, L)][0]
            loc = pk >> 18
            ov = hist[pl.ds(loc, L)]
            pos = ov[0]
            plsc.store_compressed(sbuf.at[pl.ds(pos, L)],
                                  jnp.full((L,), 0, jnp.int32) + pk,
                                  mask=lane0)
            hist[pl.ds(loc, L)] = ov + onehot0
            return 0

        lax.fori_loop(0, chunk_n, place, 0)
        pltpu.sync_copy(sbuf.at[pl.ds(0, SCAP)], lists.at[pl.ds(o, SCAP)])
        return 0

    lax.fori_loop(0, nchb, sort_chunk, 0)


def _sc_partition(dst):
    f = pl.kernel(
        _partition_body,
        out_type=(
            jax.ShapeDtypeStruct((NW * CAPL,), jnp.int32),
            jax.ShapeDtypeStruct((NW * FG,), jnp.int32),
        ),
        mesh=_mesh,
        compiler_params=_sc_params,
        scratch_types=[
            pltpu.VMEM((CE,), jnp.int32),
            pltpu.VMEM((MCAP,), jnp.int32),
            pltpu.VMEM((FG,), jnp.int32),
            pltpu.VMEM((SCAP + L,), jnp.int32),
            pltpu.VMEM((HCAP,), jnp.int32),
            pltpu.SemaphoreType.DMA,
        ],
    )
    return f(dst)


def _apply_passes(lists, counts, msg_refs, out, col_offs,
                  acc, pkwin, eidwin, cntbuf, rb0, rb1, sem0, sem1):
    w = _wid()
    lo = w * RPT
    lbase = w * CAPL
    rbufs = (rb0, rb1)
    sems = (sem0, sem1)

    pltpu.sync_copy(counts.at[pl.ds(FG * w, FG)], cntbuf)
    count = cntbuf[pl.ds(0, L)][0]
    nb = (count + GB - 1) // GB
    nwin = (nb + WB - 1) // WB

    for msg, coff in zip(msg_refs, col_offs):
        def zbody(r, _):
            for k in range(256 // L):
                acc[r, pl.ds(k * L, L)] = jnp.zeros((L,), jnp.float32)
            return 0

        lax.fori_loop(0, RPT + 8, zbody, 0)

        def win_body(win, _):
            o = pl.multiple_of(lbase + win * WCAP, FG)
            pltpu.sync_copy(lists.at[pl.ds(o, WCAP)], pkwin)
            nbw = jnp.minimum(nb - win * WB, WB)

            def unpack(v, _):
                pw = pkwin[pl.ds(v * L, L)]
                eidwin[pl.ds(v * L, L)] = pw & 0x3FFFF
                return 0

            lax.fori_loop(0, nbw * (GB // L), unpack, 0)

            def issue(g, slot):
                pltpu.async_copy(
                    msg.at[eidwin.at[pl.ds(g * GB, GB)]], rbufs[slot],
                    sems[slot])

            def wait_slot(slot):
                pltpu.make_async_copy(
                    msg.at[eidwin.at[pl.ds(0, GB)]], rbufs[slot],
                    sems[slot]).wait()

            def process(g, slot):
                rowbuf = rbufs[slot]

                def sub_j(j, _):
                    pkv = pkwin[pl.ds(g * GB + j * L, L)]
                    for jj in range(L):
                        r = pkv[jj] >> 18
                        i = j * L + jj
                        for k in range(256 // L):
                            a = acc[r, pl.ds(k * L, L)]
                            b = rowbuf[i, pl.ds(k * L, L)]
                            acc[r, pl.ds(k * L, L)] = jnp.maximum(a, b)
                    return 0

                lax.fori_loop(0, GB // L, sub_j, 0)

            issue(0, 0)

            def proc_g(g, _):
                even = lax.rem(g, 2) == 0

                @pl.when((g + 1 < nbw) & even)
                def _():
                    issue(g + 1, 1)

                @pl.when((g + 1 < nbw) & jnp.logical_not(even))
                def _():
                    issue(g + 1, 0)

                @pl.when(even)
                def _():
                    wait_slot(0)
                    process(g, 0)

                @pl.when(jnp.logical_not(even))
                def _():
                    wait_slot(1)
                    process(g, 1)

                return 0

            lax.fori_loop(0, nbw, proc_g, 0)
            return 0

        lax.fori_loop(0, nwin, win_body, 0)

        pltpu.sync_copy(acc.at[pl.ds(0, RPT), :],
                        out.at[pl.ds(lo, RPT), pl.ds(coff, 256)])


def _sc_scatter_max(lists, counts, msgs, out_cols):
    col_offs = [i * 256 for i in range(len(msgs))]

    def body(lists_ref, counts_ref, *refs):
        msg_refs = list(refs[:len(msgs)])
        out = refs[len(msgs)]
        scratch = refs[len(msgs) + 1:]
        _apply_passes(lists_ref, counts_ref, msg_refs, out, col_offs,
                      *scratch)

    f = pl.kernel(
        body,
        out_type=jax.ShapeDtypeStruct((NPAD, out_cols), jnp.float32),
        mesh=_mesh,
        compiler_params=_sc_params,
        scratch_types=[
            pltpu.VMEM((RPT + 8, 256), jnp.float32),
            pltpu.VMEM((WCAP,), jnp.int32),
            pltpu.VMEM((WCAP,), jnp.int32),
            pltpu.VMEM((FG,), jnp.int32),
            pltpu.VMEM((GB, 256), jnp.float32),
            pltpu.VMEM((GB, 256), jnp.float32),
            pltpu.SemaphoreType.DMA,
            pltpu.SemaphoreType.DMA,
        ],
    )
    return f(lists, counts, *msgs)


def _layer1_body(ef, wa, ba, wb, bb, out):
    dn = (((0,), (0,)), ((), ()))
    x = lax.dot_general(ef[...], wa[...], dn,
                        preferred_element_type=jnp.float32)
    x = x + ba[...]
    z = jnp.maximum(x, 0.0)
    out[...] = jnp.dot(z, wb[...],
                       preferred_element_type=jnp.float32) + bb[...]


def _tc_layer1(ef_t, wa, ba, wb, bb, blk=1280):
    grid = (E // blk,)
    return pl.pallas_call(
        _layer1_body,
        grid=grid,
        in_specs=[
            pl.BlockSpec((6, blk), lambda i: (0, i)),
            pl.BlockSpec((6, 256), lambda i: (0, 0)),
            pl.BlockSpec((1, 256), lambda i: (0, 0)),
            pl.BlockSpec((256, 256), lambda i: (0, 0)),
            pl.BlockSpec((1, 256), lambda i: (0, 0)),
        ],
        out_specs=pl.BlockSpec((blk, 256), lambda i: (i, 0)),
        out_shape=jax.ShapeDtypeStruct((E, 256), jnp.float32),
    )(ef_t, wa, ba, wb, bb)


def _layer2_body(hs, ef, wh, wr, ba, wb, bb, outa, outb):
    dn = (((0,), (0,)), ((), ()))
    x = jnp.dot(hs[...], wh[...], preferred_element_type=jnp.float32)
    x = x + lax.dot_general(ef[...], wr[...], dn,
                            preferred_element_type=jnp.float32)
    x = x + ba[...]
    z = jnp.maximum(x, 0.0)
    m = jnp.dot(z, wb[...], preferred_element_type=jnp.float32) + bb[...]
    outa[...] = m[:, :256]
    outb[...] = m[:, 256:]


def _tc_layer2(hs, ef_t, wh, wr, ba, wb, bb, blk=1280):
    grid = (E // blk,)
    return pl.pallas_call(
        _layer2_body,
        grid=grid,
        in_specs=[
            pl.BlockSpec((blk, 256), lambda i: (i, 0)),
            pl.BlockSpec((6, blk), lambda i: (0, i)),
            pl.BlockSpec((256, 512), lambda i: (0, 0)),
            pl.BlockSpec((6, 512), lambda i: (0, 0)),
            pl.BlockSpec((1, 512), lambda i: (0, 0)),
            pl.BlockSpec((512, 512), lambda i: (0, 0)),
            pl.BlockSpec((1, 512), lambda i: (0, 0)),
        ],
        out_specs=[
            pl.BlockSpec((blk, 256), lambda i: (i, 0)),
            pl.BlockSpec((blk, 256), lambda i: (i, 0)),
        ],
        out_shape=[
            jax.ShapeDtypeStruct((E, 256), jnp.float32),
            jax.ShapeDtypeStruct((E, 256), jnp.float32),
        ],
    )(hs, ef_t, wh, wr, ba, wb, bb)


def kernel(pos, edge_index, batch, W1a, b1a, W1b, b1b, W2a, b2a, W2b, b2b):
    src = edge_index[0]
    dst = edge_index[1]
    srcp = jnp.pad(src, (0, GB))

    ef_t = _sc_geom(pos[:, 0], pos[:, 1], pos[:, 2], src, dst)
    lists, counts = _sc_partition(dst)

    m1 = _tc_layer1(ef_t, W1a, b1a.reshape(1, 256), W1b, b1b.reshape(1, 256))
    h1 = _sc_scatter_max(lists, counts, [m1], 256)

    hs = _sc_gather(h1, srcp)

    w2r6 = jnp.zeros((6, 512), jnp.float32).at[3:6].set(W2a[256:259])
    m2a, m2b = _tc_layer2(hs, ef_t, W2a[:256], w2r6,
                          b2a.reshape(1, 512), W2b, b2b.reshape(1, 512))
    out = _sc_scatter_max(lists, counts, [m2a, m2b], 512)
    return out[:N]


# speedup vs baseline: 1.8724x; 1.1232x over previous
import jax
import jax.numpy as jnp
from jax import lax
from jax.experimental import pallas as pl
from jax.experimental.pallas import tpu as pltpu
from jax.experimental.pallas import tpu_sc as plsc

N = 10000
E = 160000
NC = 2
NS = 16
NW = NC * NS
L = 16
ER = E // L

EPT_G = 4992
ROWS_G = EPT_G // L
TAIL_BASE = NW * EPT_G

EPT = E // NW
GB = 64

RPT = 320
NPAD = NW * RPT
CE = 4096
NCHUNK = E // CE
TE = E - NCHUNK * CE
FG = 128
MCAP = CE + 2 * FG
CAPL = E + FG
WB = 40
WCAP = WB * GB

_mesh = plsc.VectorSubcoreMesh(core_axis_name="c", subcore_axis_name="s")
_sc_params = pltpu.CompilerParams(needs_layout_passes=False)


def _wid():
    return lax.axis_index("s") * NC + lax.axis_index("c")


def _iota():
    return lax.iota(jnp.int32, L)


def _geom_range(src, dst, posx, posy, posz, ef_out,
                sidx, didx, sxb, dxb, efbuf, sem, base, cnt):
    pltpu.sync_copy(src.at[pl.ds(base, cnt)], sidx.at[pl.ds(0, cnt)])
    pltpu.sync_copy(dst.at[pl.ds(base, cnt)], didx.at[pl.ds(0, cnt)])
    for c, table in enumerate((posx, posy, posz)):
        pltpu.async_copy(table.at[sidx.at[pl.ds(0, cnt)]],
                         sxb.at[pl.ds(0, cnt)], sem).wait()
        pltpu.async_copy(table.at[didx.at[pl.ds(0, cnt)]],
                         dxb.at[pl.ds(0, cnt)], sem).wait()

        def body(v, _):
            sv = sxb[pl.ds(v * L, L)]
            dv = dxb[pl.ds(v * L, L)]
            efbuf[c, pl.ds(v * L, L)] = sv
            efbuf[3 + c, pl.ds(v * L, L)] = sv - dv
            return 0

        lax.fori_loop(0, cnt // L, body, 0)
    pltpu.sync_copy(efbuf.at[:, pl.ds(0, cnt)],
                    ef_out.at[:, pl.ds(base, cnt)])


def _geom_body(posx, posy, posz, src, dst, ef_out,
               sidx, didx, sxb, dxb, efbuf, sem):
    w = _wid()
    _geom_range(src, dst, posx, posy, posz, ef_out,
                sidx, didx, sxb, dxb, efbuf, sem, w * EPT_G, EPT_G)

    @pl.when(w < 2)
    def _():
        _geom_range(src, dst, posx, posy, posz, ef_out,
                    sidx, didx, sxb, dxb, efbuf, sem,
                    TAIL_BASE + w * 128, 128)


def _sc_geom(posx, posy, posz, src, dst):
    f = pl.kernel(
        _geom_body,
        out_type=jax.ShapeDtypeStruct((6, E), jnp.float32),
        mesh=_mesh,
        compiler_params=_sc_params,
        scratch_types=[
            pltpu.VMEM((EPT_G,), jnp.int32),
            pltpu.VMEM((EPT_G,), jnp.int32),
            pltpu.VMEM((EPT_G,), jnp.float32),
            pltpu.VMEM((EPT_G,), jnp.float32),
            pltpu.VMEM((6, EPT_G), jnp.float32),
            pltpu.SemaphoreType.DMA,
        ],
    )
    return f(posx, posy, posz, src, dst)


def _gather_body(table, src, out, idxbuf, rb0, rb1, gs0, gs1, os0, os1):
    w = _wid()
    base = w * EPT
    nfull = EPT // GB
    tail = EPT - nfull * GB
    pltpu.sync_copy(src.at[pl.ds(base, EPT + GB)], idxbuf)
    rbufs = (rb0, rb1)
    gsems = (gs0, gs1)
    osems = (os0, os1)

    def issue(g, slot):
        pltpu.async_copy(table.at[idxbuf.at[pl.ds(g * GB, GB)]],
                         rbufs[slot], gsems[slot])

    def wait_gather(slot):
        pltpu.make_async_copy(table.at[idxbuf.at[pl.ds(0, GB)]],
                              rbufs[slot], gsems[slot]).wait()

    def copy_out(g, slot):
        pltpu.async_copy(rbufs[slot], out.at[pl.ds(base + g * GB, GB)],
                         osems[slot])

    def wait_out(slot):
        pltpu.make_async_copy(rbufs[slot], out.at[pl.ds(base, GB)],
                              osems[slot]).wait()

    issue(0, 0)

    def body(g, _):
        even = lax.rem(g, 2) == 0

        @pl.when((g + 1 < nfull) & even)
        def _():
            @pl.when(g + 1 >= 2)
            def _():
                wait_out(1)
            issue(g + 1, 1)

        @pl.when((g + 1 < nfull) & jnp.logical_not(even))
        def _():
            @pl.when(g + 1 >= 2)
            def _():
                wait_out(0)
            issue(g + 1, 0)

        @pl.when(even)
        def _():
            wait_gather(0)
            copy_out(g, 0)

        @pl.when(jnp.logical_not(even))
        def _():
            wait_gather(1)
            copy_out(g, 1)

        return 0

    lax.fori_loop(0, nfull, body, 0)
    wait_out((nfull - 2) % 2)
    wait_out((nfull - 1) % 2)
    pltpu.async_copy(table.at[idxbuf.at[pl.ds(nfull * GB, GB)]], rb0,
                     gs0).wait()
    pltpu.sync_copy(rb0.at[pl.ds(0, tail)],
                    out.at[pl.ds(base + nfull * GB, tail)])


def _sc_gather(table, src):
    f = pl.kernel(
        _gather_body,
        out_type=jax.ShapeDtypeStruct((E, 256), jnp.float32),
        mesh=_mesh,
        compiler_params=_sc_params,
        scratch_types=[
            pltpu.VMEM((EPT + GB,), jnp.int32),
            pltpu.VMEM((GB, 256), jnp.float32),
            pltpu.VMEM((GB, 256), jnp.float32),
            pltpu.SemaphoreType.DMA,
            pltpu.SemaphoreType.DMA,
            pltpu.SemaphoreType.DMA,
            pltpu.SemaphoreType.DMA,
        ],
    )
    return f(table, src)


def _partition_body(dst, lists, counts, dstbuf, pkbuf, cntbuf, sem):
    w = _wid()
    lo = w * RPT
    lbase = w * CAPL
    junk = jnp.full((L,), RPT << 18, jnp.int32)

    def do_chunk(ebase, ne, state):
        gp, carry = state
        pltpu.sync_copy(dst.at[pl.ds(ebase, ne)], dstbuf.at[pl.ds(0, ne)])
        base_eid = jnp.full((L,), ebase, jnp.int32) + _iota()

        def scan_v(v, off):
            d = dstbuf[pl.ds(v * L, L)]
            m = (d >= lo) & (d < lo + RPT)
            pk = ((d - lo) << 18) | (base_eid + v * L)
            plsc.store_compressed(pkbuf.at[pl.ds(off, L)], pk, mask=m)
            cnt = plsc.all_reduce_population_count(m)
            return off + cnt[0]

        total = lax.fori_loop(0, ne // L, scan_v, carry)
        nfull = total // FG

        def flush(b, _):
            o = pl.multiple_of(lbase + gp + b * FG, FG)
            pltpu.sync_copy(pkbuf.at[pl.ds(b * FG, FG)],
                            lists.at[pl.ds(o, FG)])
            return 0

        lax.fori_loop(0, nfull, flush, 0)
        tsrc = nfull * FG
        for t in range(FG // L):
            v = pkbuf[pl.ds(tsrc + t * L, L)]
            pkbuf[pl.ds(t * L, L)] = v
        return gp + nfull * FG, total - nfull * FG

    def chunk_body(c, state):
        return do_chunk(c * CE, CE, state)

    state = lax.fori_loop(0, NCHUNK, chunk_body, (0, 0))
    gp, carry = do_chunk(NCHUNK * CE, TE, state)

    for t in range(FG // L):
        pkbuf[pl.ds(carry + t * L, L)] = junk
    pltpu.sync_copy(pkbuf.at[pl.ds(0, FG)],
                    lists.at[pl.ds(pl.multiple_of(lbase + gp, FG), FG)])
    count = gp + carry
    cvec = jnp.full((L,), 0, jnp.int32) + count
    for t in range(FG // L):
        cntbuf[pl.ds(t * L, L)] = cvec
    pltpu.sync_copy(cntbuf, counts.at[pl.ds(FG * w, FG)])


def _sc_partition(dst):
    f = pl.kernel(
        _partition_body,
        out_type=(
            jax.ShapeDtypeStruct((NW * CAPL,), jnp.int32),
            jax.ShapeDtypeStruct((NW * FG,), jnp.int32),
        ),
        mesh=_mesh,
        compiler_params=_sc_params,
        scratch_types=[
            pltpu.VMEM((CE,), jnp.int32),
            pltpu.VMEM((MCAP,), jnp.int32),
            pltpu.VMEM((FG,), jnp.int32),
            pltpu.SemaphoreType.DMA,
        ],
    )
    return f(dst)


def _apply_passes(lists, counts, msg_refs, out, col_offs,
                  acc, pkwin, eidwin, cntbuf, rb0, rb1, sem0, sem1):
    w = _wid()
    lo = w * RPT
    lbase = w * CAPL
    rbufs = (rb0, rb1)
    sems = (sem0, sem1)

    pltpu.sync_copy(counts.at[pl.ds(FG * w, FG)], cntbuf)
    count = cntbuf[pl.ds(0, L)][0]
    nb = (count + GB - 1) // GB
    nwin = (nb + WB - 1) // WB

    for msg, coff in zip(msg_refs, col_offs):
        def zbody(r, _):
            for k in range(256 // L):
                acc[r, pl.ds(k * L, L)] = jnp.zeros((L,), jnp.float32)
            return 0

        lax.fori_loop(0, RPT + 8, zbody, 0)

        def win_body(win, _):
            o = pl.multiple_of(lbase + win * WCAP, FG)
            pltpu.sync_copy(lists.at[pl.ds(o, WCAP)], pkwin)
            nbw = jnp.minimum(nb - win * WB, WB)

            def unpack(v, _):
                pw = pkwin[pl.ds(v * L, L)]
                eidwin[pl.ds(v * L, L)] = pw & 0x3FFFF
                return 0

            lax.fori_loop(0, nbw * (GB // L), unpack, 0)

            def issue(g, slot):
                pltpu.async_copy(
                    msg.at[eidwin.at[pl.ds(g * GB, GB)]], rbufs[slot],
                    sems[slot])

            def wait_slot(slot):
                pltpu.make_async_copy(
                    msg.at[eidwin.at[pl.ds(0, GB)]], rbufs[slot],
                    sems[slot]).wait()

            def process(g, slot):
                rowbuf = rbufs[slot]

                def sub_j(j, _):
                    pkv = pkwin[pl.ds(g * GB + j * L, L)]
                    for jj in range(L):
                        r = pkv[jj] >> 18
                        i = j * L + jj
                        for k in range(256 // L):
                            a = acc[r, pl.ds(k * L, L)]
                            b = rowbuf[i, pl.ds(k * L, L)]
                            acc[r, pl.ds(k * L, L)] = jnp.maximum(a, b)
                    return 0

                lax.fori_loop(0, GB // L, sub_j, 0)

            issue(0, 0)

            def proc_g(g, _):
                even = lax.rem(g, 2) == 0

                @pl.when((g + 1 < nbw) & even)
                def _():
                    issue(g + 1, 1)

                @pl.when((g + 1 < nbw) & jnp.logical_not(even))
                def _():
                    issue(g + 1, 0)

                @pl.when(even)
                def _():
                    wait_slot(0)
                    process(g, 0)

                @pl.when(jnp.logical_not(even))
                def _():
                    wait_slot(1)
                    process(g, 1)

                return 0

            lax.fori_loop(0, nbw, proc_g, 0)
            return 0

        lax.fori_loop(0, nwin, win_body, 0)

        pltpu.sync_copy(acc.at[pl.ds(0, RPT), :],
                        out.at[pl.ds(lo, RPT), pl.ds(coff, 256)])


def _sc_scatter_max(lists, counts, msgs, out_cols):
    col_offs = [i * 256 for i in range(len(msgs))]

    def body(lists_ref, counts_ref, *refs):
        msg_refs = list(refs[:len(msgs)])
        out = refs[len(msgs)]
        scratch = refs[len(msgs) + 1:]
        _apply_passes(lists_ref, counts_ref, msg_refs, out, col_offs,
                      *scratch)

    f = pl.kernel(
        body,
        out_type=jax.ShapeDtypeStruct((NPAD, out_cols), jnp.float32),
        mesh=_mesh,
        compiler_params=_sc_params,
        scratch_types=[
            pltpu.VMEM((RPT + 8, 256), jnp.float32),
            pltpu.VMEM((WCAP,), jnp.int32),
            pltpu.VMEM((WCAP,), jnp.int32),
            pltpu.VMEM((FG,), jnp.int32),
            pltpu.VMEM((GB, 256), jnp.float32),
            pltpu.VMEM((GB, 256), jnp.float32),
            pltpu.SemaphoreType.DMA,
            pltpu.SemaphoreType.DMA,
        ],
    )
    return f(lists, counts, *msgs)


def _layer1_body(ef, wa, ba, wb, bb, out):
    dn = (((0,), (0,)), ((), ()))
    x = lax.dot_general(ef[...], wa[...], dn,
                        preferred_element_type=jnp.float32)
    x = x + ba[...]
    z = jnp.maximum(x, 0.0)
    out[...] = jnp.dot(z, wb[...],
                       preferred_element_type=jnp.float32) + bb[...]


def _tc_layer1(ef_t, wa, ba, wb, bb, blk=1280):
    grid = (E // blk,)
    return pl.pallas_call(
        _layer1_body,
        grid=grid,
        in_specs=[
            pl.BlockSpec((6, blk), lambda i: (0, i)),
            pl.BlockSpec((6, 256), lambda i: (0, 0)),
            pl.BlockSpec((1, 256), lambda i: (0, 0)),
            pl.BlockSpec((256, 256), lambda i: (0, 0)),
            pl.BlockSpec((1, 256), lambda i: (0, 0)),
        ],
        out_specs=pl.BlockSpec((blk, 256), lambda i: (i, 0)),
        out_shape=jax.ShapeDtypeStruct((E, 256), jnp.float32),
    )(ef_t, wa, ba, wb, bb)


def _layer2_body(hs, ef, wh, wr, ba, wb, bb, outa, outb):
    dn = (((0,), (0,)), ((), ()))
    x = jnp.dot(hs[...], wh[...], preferred_element_type=jnp.float32)
    x = x + lax.dot_general(ef[...], wr[...], dn,
                            preferred_element_type=jnp.float32)
    x = x + ba[...]
    z = jnp.maximum(x, 0.0)
    m = jnp.dot(z, wb[...], preferred_element_type=jnp.float32) + bb[...]
    outa[...] = m[:, :256]
    outb[...] = m[:, 256:]


def _tc_layer2(hs, ef_t, wh, wr, ba, wb, bb, blk=1280):
    grid = (E // blk,)
    return pl.pallas_call(
        _layer2_body,
        grid=grid,
        in_specs=[
            pl.BlockSpec((blk, 256), lambda i: (i, 0)),
            pl.BlockSpec((6, blk), lambda i: (0, i)),
            pl.BlockSpec((256, 512), lambda i: (0, 0)),
            pl.BlockSpec((6, 512), lambda i: (0, 0)),
            pl.BlockSpec((1, 512), lambda i: (0, 0)),
            pl.BlockSpec((512, 512), lambda i: (0, 0)),
            pl.BlockSpec((1, 512), lambda i: (0, 0)),
        ],
        out_specs=[
            pl.BlockSpec((blk, 256), lambda i: (i, 0)),
            pl.BlockSpec((blk, 256), lambda i: (i, 0)),
        ],
        out_shape=[
            jax.ShapeDtypeStruct((E, 256), jnp.float32),
            jax.ShapeDtypeStruct((E, 256), jnp.float32),
        ],
    )(hs, ef_t, wh, wr, ba, wb, bb)


def kernel(pos, edge_index, batch, W1a, b1a, W1b, b1b, W2a, b2a, W2b, b2b):
    src = edge_index[0]
    dst = edge_index[1]
    srcp = jnp.pad(src, (0, GB))

    ef_t = _sc_geom(pos[:, 0], pos[:, 1], pos[:, 2], src, dst)
    lists, counts = _sc_partition(dst)

    m1 = _tc_layer1(ef_t, W1a, b1a.reshape(1, 256), W1b, b1b.reshape(1, 256))
    h1 = _sc_scatter_max(lists, counts, [m1], 256)

    hs = _sc_gather(h1, srcp)

    w2r6 = jnp.zeros((6, 512), jnp.float32).at[3:6].set(W2a[256:259])
    m2a, m2b = _tc_layer2(hs, ef_t, W2a[:256], w2r6,
                          b2a.reshape(1, 512), W2b, b2b.reshape(1, 512))
    out = _sc_scatter_max(lists, counts, [m2a, m2b], 512)
    return out[:N]
